# Initial kernel scaffold; baseline (speedup 1.0000x reference)
import jax
import jax.numpy as jnp
from jax.experimental import pallas as pl


def kernel(hidden_states, weight, e_score_correction_bias):
    raise NotImplementedError("write your pallas kernel here")


# speedup vs baseline: 1.1408x; 1.1408x over previous
import functools

import jax
import jax.numpy as jnp
from jax.experimental import pallas as pl

_HIDDEN = 2048
_E = 64
_TOP_K = 8
_N_GROUP = 8
_PER_GROUP = _E // _N_GROUP
_TOPK_GROUP = 4
_BLOCK_T = 512

_NEG_INF = float("-inf")


def _router_kernel(x_ref, wt_ref, b_ref, idx_ref, w_ref):
    x = x_ref[...]
    logits = jax.lax.dot_general(
        x,
        wt_ref[...],
        (((1,), (0,)), ((), ())),
        preferred_element_type=jnp.float32,
        precision=jax.lax.Precision.DEFAULT,
    )
    scores = jax.nn.sigmoid(logits)
    sfc = scores + b_ref[...]

    bt = x.shape[0]

    iota_pg = jax.lax.broadcasted_iota(jnp.int32, (bt, _PER_GROUP), 1)
    group_cols = []
    for g in range(_N_GROUP):
        grp = sfc[:, g * _PER_GROUP : (g + 1) * _PER_GROUP]
        m1 = jnp.max(grp, axis=1, keepdims=True)
        eq = grp == m1
        first = jnp.min(
            jnp.where(eq, iota_pg, _PER_GROUP), axis=1, keepdims=True
        )
        grp2 = jnp.where(iota_pg == first, _NEG_INF, grp)
        m2 = jnp.max(grp2, axis=1, keepdims=True)
        group_cols.append(m1 + m2)
    group_scores = jnp.concatenate(group_cols, axis=1)

    iota_g = jax.lax.broadcasted_iota(jnp.int32, (bt, _N_GROUP), 1)
    sel = jnp.zeros((bt, _N_GROUP), dtype=jnp.bool_)
    gwork = group_scores
    for _ in range(_TOPK_GROUP):
        m = jnp.max(gwork, axis=1, keepdims=True)
        eq = gwork == m
        first = jnp.min(jnp.where(eq, iota_g, _N_GROUP), axis=1, keepdims=True)
        pick = iota_g == first
        sel = jnp.logical_or(sel, pick)
        gwork = jnp.where(pick, _NEG_INF, gwork)

    self_f = jnp.where(sel, 1.0, 0.0)
    mask_cols = []
    for g in range(_N_GROUP):
        mask_cols.append(
            jnp.broadcast_to(self_f[:, g : g + 1], (bt, _PER_GROUP))
        )
    mask64 = jnp.concatenate(mask_cols, axis=1)
    masked = jnp.where(mask64 > 0.0, sfc, 0.0)

    iota_e = jax.lax.broadcasted_iota(jnp.int32, (bt, _E), 1)
    work = masked
    idx_cols = []
    w_cols = []
    for _ in range(_TOP_K):
        m = jnp.max(work, axis=1, keepdims=True)
        eq = work == m
        first = jnp.min(jnp.where(eq, iota_e, _E), axis=1, keepdims=True)
        onehot = iota_e == first
        idx_cols.append(first)
        w_cols.append(
            jnp.sum(jnp.where(onehot, scores, 0.0), axis=1, keepdims=True)
        )
        work = jnp.where(onehot, _NEG_INF, work)

    idx_all = jnp.concatenate(idx_cols, axis=1)
    w_all = jnp.concatenate(w_cols, axis=1)
    denom = jnp.sum(w_all, axis=1, keepdims=True) + 1e-20
    idx_ref[...] = idx_all
    w_ref[...] = w_all / denom


@jax.jit
def kernel(hidden_states, weight, e_score_correction_bias):
    tokens = hidden_states.shape[0]
    wt = weight.astype(jnp.float32).T
    bias = e_score_correction_bias.astype(jnp.float32).reshape(1, _E)
    grid = (tokens // _BLOCK_T,)
    idx, w = pl.pallas_call(
        _router_kernel,
        grid=grid,
        in_specs=[
            pl.BlockSpec((_BLOCK_T, _HIDDEN), lambda i: (i, 0)),
            pl.BlockSpec((_HIDDEN, _E), lambda i: (0, 0)),
            pl.BlockSpec((1, _E), lambda i: (0, 0)),
        ],
        out_specs=[
            pl.BlockSpec((_BLOCK_T, _TOP_K), lambda i: (i, 0)),
            pl.BlockSpec((_BLOCK_T, _TOP_K), lambda i: (i, 0)),
        ],
        out_shape=[
            jax.ShapeDtypeStruct((tokens, _TOP_K), jnp.int32),
            jax.ShapeDtypeStruct((tokens, _TOP_K), jnp.float32),
        ],
    )(hidden_states.astype(jnp.float32), wt, bias)
    return idx, w


# speedup vs baseline: 5.4722x; 4.7970x over previous
import jax
import jax.numpy as jnp
from jax.experimental import pallas as pl

_HIDDEN = 2048
_E = 64
_TOP_K = 8
_N_GROUP = 8
_PER_GROUP = _E // _N_GROUP
_TOPK_GROUP = 4
_BLOCK_T = 512

_NEG_INF = float("-inf")


def _router_kernel(x_ref, wt_ref, b_ref, idx_ref, w_ref):
    x = x_ref[...]
    logits = jax.lax.dot_general(
        x,
        wt_ref[...],
        (((1,), (0,)), ((), ())),
        preferred_element_type=jnp.float32,
        precision=jax.lax.Precision.DEFAULT,
    )
    lt = jax.lax.transpose(logits, (1, 0))
    bt = lt.shape[1]
    scores = jax.nn.sigmoid(lt)
    sfc = scores + b_ref[...]

    iota_pg = jax.lax.broadcasted_iota(jnp.int32, (_PER_GROUP, bt), 0).astype(jnp.float32)
    group_rows = []
    for g in range(_N_GROUP):
        grp = sfc[g * _PER_GROUP : (g + 1) * _PER_GROUP, :]
        m1 = jnp.max(grp, axis=0, keepdims=True)
        first = jnp.min(
            jnp.where(grp == m1, iota_pg, float(_PER_GROUP)),
            axis=0,
            keepdims=True,
        )
        m2 = jnp.max(
            jnp.where(iota_pg == first, _NEG_INF, grp), axis=0, keepdims=True
        )
        group_rows.append(m1 + m2)
    group_scores = jnp.concatenate(group_rows, axis=0)

    iota_g = jax.lax.broadcasted_iota(jnp.int32, (_N_GROUP, bt), 0).astype(jnp.float32)
    sel = jnp.zeros((_N_GROUP, bt), dtype=jnp.float32)
    gwork = group_scores
    for _ in range(_TOPK_GROUP):
        m = jnp.max(gwork, axis=0, keepdims=True)
        first = jnp.min(
            jnp.where(gwork == m, iota_g, float(_N_GROUP)),
            axis=0,
            keepdims=True,
        )
        pick = iota_g == first
        sel = jnp.where(pick, 1.0, sel)
        gwork = jnp.where(pick, _NEG_INF, gwork)

    mask_rows = []
    for g in range(_N_GROUP):
        mask_rows.append(jnp.broadcast_to(sel[g : g + 1, :], (_PER_GROUP, bt)))
    mask64 = jnp.concatenate(mask_rows, axis=0)
    masked = jnp.where(mask64 > 0.0, sfc, 0.0)

    iota_e = jax.lax.broadcasted_iota(jnp.int32, (_E, bt), 0).astype(jnp.float32)
    work = masked
    idx_rows = []
    w_rows = []
    for _ in range(_TOP_K):
        m = jnp.max(work, axis=0, keepdims=True)
        first = jnp.min(
            jnp.where(work == m, iota_e, float(_E)), axis=0, keepdims=True
        )
        onehot = iota_e == first
        idx_rows.append(first)
        w_rows.append(
            jnp.sum(jnp.where(onehot, scores, 0.0), axis=0, keepdims=True)
        )
        work = jnp.where(onehot, _NEG_INF, work)

    idxf = jnp.concatenate(idx_rows, axis=0)
    w_all = jnp.concatenate(w_rows, axis=0)
    denom = jnp.sum(w_all, axis=0, keepdims=True) + 1e-20
    wn = w_all / denom
    idx_ref[...] = jax.lax.transpose(idxf.astype(jnp.int32), (1, 0))
    w_ref[...] = jax.lax.transpose(wn, (1, 0))


@jax.jit
def kernel(hidden_states, weight, e_score_correction_bias):
    tokens = hidden_states.shape[0]
    wt = weight.astype(jnp.float32).T
    bias = e_score_correction_bias.astype(jnp.float32).reshape(_E, 1)
    grid = (tokens // _BLOCK_T,)
    idx, w = pl.pallas_call(
        _router_kernel,
        grid=grid,
        in_specs=[
            pl.BlockSpec((_BLOCK_T, _HIDDEN), lambda i: (i, 0)),
            pl.BlockSpec((_HIDDEN, _E), lambda i: (0, 0)),
            pl.BlockSpec((_E, 1), lambda i: (0, 0)),
        ],
        out_specs=[
            pl.BlockSpec((_BLOCK_T, _TOP_K), lambda i: (i, 0)),
            pl.BlockSpec((_BLOCK_T, _TOP_K), lambda i: (i, 0)),
        ],
        out_shape=[
            jax.ShapeDtypeStruct((tokens, _TOP_K), jnp.int32),
            jax.ShapeDtypeStruct((tokens, _TOP_K), jnp.float32),
        ],
    )(hidden_states.astype(jnp.float32), wt, bias)
    return idx, w
